# baseline (speedup 1.0000x reference)
import jax
import jax.numpy as jnp
from jax.experimental import pallas as pl


def _pe_kernel(feats_ref, a_ref, b_ref, glb_ref, out_ref):
    cc = feats_ref.shape[1]
    hw = feats_ref.shape[2]
    h = b_ref.shape[0]
    w = a_ref.shape[0]

    x = feats_ref[0]
    xt = jnp.transpose(x, (1, 0))

    a = a_ref[...]
    bb = b_ref[...]
    pe = (jnp.broadcast_to(a[None, :, :], (h, w, cc)).reshape(hw, cc)
          + jnp.broadcast_to(bb[:, None, :], (h, w, cc)).reshape(hw, cc))

    out_ref[0, 1:, :] = xt + pe
    out_ref[0, 0:1, :] = glb_ref[...]


def kernel(feats, pe_x_table, pe_y_table, glb_table):
    b, c, h, w = feats.shape
    hw = h * w
    dim = pe_x_table.shape[1]
    feats2 = feats.reshape(b, c, hw)

    a_full = jnp.concatenate(
        [pe_x_table, jnp.zeros((w, c - dim), pe_x_table.dtype)], axis=1)
    b_full = jnp.concatenate(
        [jnp.zeros((h, c - dim), pe_y_table.dtype), pe_y_table], axis=1)

    cc = 128
    ncc = c // cc

    out = pl.pallas_call(
        _pe_kernel,
        grid=(ncc, b),
        in_specs=[
            pl.BlockSpec((1, cc, hw), lambda j, i: (i, j, 0)),
            pl.BlockSpec((w, cc), lambda j, i: (0, j)),
            pl.BlockSpec((h, cc), lambda j, i: (0, j)),
            pl.BlockSpec((1, cc), lambda j, i: (0, j)),
        ],
        out_specs=pl.BlockSpec((1, 1 + hw, cc), lambda j, i: (i, 0, j)),
        out_shape=jax.ShapeDtypeStruct((b, 1 + hw, c), feats.dtype),
    )(feats2, a_full, b_full, glb_table)
    return out
